# baseline (speedup 1.0000x reference)
import functools

import jax
import jax.numpy as jnp
from jax import lax
from jax.experimental import pallas as pl
from jax.experimental.pallas import tpu as pltpu
from jax.experimental.pallas import tpu_sc as plsc

F = 26
D = 16
ROWS = 1000
TOTAL = F * ROWS
B = 4096
NC, NS, L = 2, 16, 16
NW = NC * NS
NPAIRS = F * (F - 1) // 2
GRPS = B // L
RPT = B // NS
BANDW = 1152
TOTALP = 26112
DP = D // 2

_ROW_START = [r * (F - 1) - r * (r - 1) // 2 for r in range(F - 1)]

_mesh = plsc.VectorSubcoreMesh(core_axis_name="c", subcore_axis_name="s")


@functools.partial(
    pl.kernel,
    out_type=(jax.ShapeDtypeStruct((NW, B), jnp.float32),
              jax.ShapeDtypeStruct((NC, F * B), jnp.int32)),
    mesh=_mesh,
    scratch_types=[
        pltpu.VMEM((DP, BANDW), jnp.int32),
        pltpu.VMEM((DP, BANDW), jnp.int32),
        pltpu.VMEM((DP, BANDW), jnp.int32),
        pltpu.VMEM((DP, BANDW), jnp.int32),
        pltpu.VMEM((B,), jnp.int32),
        pltpu.VMEM((B,), jnp.int32),
        pltpu.VMEM((B,), jnp.int32),
        pltpu.VMEM((B,), jnp.int32),
        pltpu.VMEM((B,), jnp.float32),
        pltpu.VMEM((ROWS,), jnp.float32),
        pltpu.VMEM((RPT * F,), jnp.int32),
        pltpu.VMEM((F * RPT,), jnp.int32),
        pltpu.SemaphoreType.DMA,
        pltpu.SemaphoreType.DMA,
    ],
    compiler_params=pltpu.CompilerParams(needs_layout_passes=False),
)
def _sc_ffm(ffm2, x_flat, lin_flat, out_ref, xt_hbm,
            band_a0, band_b0, band_a1, band_b1,
            x_a0, x_b0, x_a1, x_b1, partial, lin_band,
            x_rows, xt_chunk, sem0, sem1):
    cid = lax.axis_index("c")
    sid = lax.axis_index("s")
    wid = sid * NC + cid

    pltpu.sync_copy(x_flat.at[pl.ds(sid * (RPT * F), RPT * F)], x_rows)

    lanes = lax.iota(jnp.int32, L)

    @plsc.parallel_loop(0, F * (RPT // L), unroll=4)
    def t_body(q):
        f = q >> 4
        gg = q & 15
        idx = (gg * L + lanes) * F + f
        v = plsc.load_gather(x_rows, [idx])
        xt_chunk[pl.ds(f * RPT + gg * L, L)] = v

    for f in range(F):
        pltpu.sync_copy(xt_chunk.at[pl.ds(f * RPT, RPT)],
                        xt_hbm.at[cid, pl.ds(f * B + sid * RPT, RPT)])
    plsc.subcore_barrier()

    @plsc.parallel_loop(0, GRPS, unroll=4)
    def zero_body(g):
        partial[pl.ds(g * L, L)] = jnp.zeros((L,), jnp.float32)

    @pl.when(wid < F)
    def _():
        pltpu.sync_copy(lin_flat.at[pl.ds(wid * ROWS, ROWS)], lin_band)
        pltpu.sync_copy(xt_hbm.at[cid, pl.ds(wid * B, B)], x_a0)

        @plsc.parallel_loop(0, GRPS, unroll=4)
        def lin_body(g):
            base = g * L
            xv = x_a0[pl.ds(base, L)]
            lv = plsc.load_gather(lin_band, [xv])
            partial[pl.ds(base, L)] = partial[pl.ds(base, L)] + lv

    def _decode(k):
        p = wid + NW * k
        i_sc = jnp.int32(0)
        si_sc = jnp.int32(0)
        for r in range(1, F - 1):
            sr = _ROW_START[r]
            cond = p >= sr
            i_sc = jnp.where(cond, jnp.int32(r), i_sc)
            si_sc = jnp.where(cond, jnp.int32(sr), si_sc)
        j_sc = p - si_sc + i_sc + 1
        c0_a = pl.multiple_of(lax.div(i_sc * ROWS, 128) * 128, 128)
        c0_b = pl.multiple_of(lax.div(j_sc * ROWS, 128) * 128, 128)
        r_a = pl.multiple_of(j_sc * DP, 8)
        r_b = pl.multiple_of(i_sc * DP, 8)
        return i_sc, j_sc, c0_a, c0_b, r_a, r_b

    def _issue(k, ba, bb, xa, xb, sem):
        i_sc, j_sc, c0_a, c0_b, r_a, r_b = _decode(k)
        return (
            pltpu.async_copy(
                ffm2.at[pl.ds(r_a, DP), pl.ds(c0_a, BANDW)], ba, sem),
            pltpu.async_copy(
                ffm2.at[pl.ds(r_b, DP), pl.ds(c0_b, BANDW)], bb, sem),
            pltpu.async_copy(xt_hbm.at[cid, pl.ds(i_sc * B, B)], xa, sem),
            pltpu.async_copy(xt_hbm.at[cid, pl.ds(j_sc * B, B)], xb, sem),
        )

    def _compute(k, ba, bb, xa, xb, w):
        i_sc, j_sc, c0_a, c0_b, _, _ = _decode(k)
        sh_a = i_sc * ROWS - c0_a
        sh_b = j_sc * ROWS - c0_b

        @plsc.parallel_loop(0, GRPS, unroll=2)
        def grp_body(g):
            base = g * L
            ra = xa[pl.ds(base, L)] + sh_a
            rb = xb[pl.ds(base, L)] + sh_b
            accs = [jnp.zeros((L,), jnp.float32) for _ in range(4)]
            for t in range(DP):
                tv = jnp.full((L,), t, jnp.int32)
                va = plsc.load_gather(ba, [tv, ra])
                vb = plsc.load_gather(bb, [tv, rb])
                pa = plsc.bitcast(va, jnp.bfloat16)
                pb = plsc.bitcast(vb, jnp.bfloat16)
                u0, u1 = plsc.unpack(pa * pb,
                                     format=plsc.PackFormat.INTERLEAVED)
                accs[t % 4] = accs[t % 4] + (u0 + u1)
            acc = (accs[0] + accs[1]) + (accs[2] + accs[3])
            partial[pl.ds(base, L)] = partial[pl.ds(base, L)] + acc * w

    slots = [(band_a0, band_b0, x_a0, x_b0, sem0),
             (band_a1, band_b1, x_a1, x_b1, sem1)]
    n_k = jnp.where(wid < NPAIRS - 10 * NW, 11, 10)
    maxk = 11
    kk = [jnp.minimum(jnp.int32(k), n_k - 1) for k in range(maxk)]
    ws = [jnp.where(jnp.int32(k) < n_k, jnp.float32(1), jnp.float32(0))
          for k in range(maxk)]

    handles = {0: _issue(kk[0], *slots[0])}
    for k in range(maxk):
        if k + 1 < maxk:
            handles[k + 1] = _issue(kk[k + 1], *slots[(k + 1) % 2])
        for h in handles.pop(k):
            h.wait()
        _compute(kk[k], *slots[k % 2][:4], ws[k])

    pltpu.sync_copy(partial, out_ref.at[wid])


def _combine_body(p_ref, b_ref, o_ref):
    o_ref[...] = jnp.sum(p_ref[...], axis=0, keepdims=True) + b_ref[...]


_combine = pl.pallas_call(
    _combine_body,
    out_shape=jax.ShapeDtypeStruct((1, B), jnp.float32),
)


def kernel(x, linear_w, linear_b, ffm_w):
    x_flat = x.reshape(-1)
    lin_flat = linear_w.reshape(-1)
    fp = ffm_w.transpose(0, 2, 1).astype(jnp.bfloat16)
    pk = jax.lax.bitcast_convert_type(
        jnp.stack([fp[:, 0::2, :], fp[:, 1::2, :]], axis=-1), jnp.int32)
    ffm2 = jnp.pad(pk.reshape(F * DP, TOTAL),
                   ((0, 0), (0, TOTALP - TOTAL)))
    partials, _ = _sc_ffm(ffm2, x_flat, lin_flat)
    out = _combine(partials, linear_b.reshape(1, 1))
    return out.reshape(B, 1)


# speedup vs baseline: 2.1383x; 2.1383x over previous
import functools

import jax
import jax.numpy as jnp
from jax import lax
from jax.experimental import pallas as pl
from jax.experimental.pallas import tpu as pltpu
from jax.experimental.pallas import tpu_sc as plsc

F = 26
D = 16
ROWS = 1000
TOTAL = F * ROWS
B = 4096
NC, NS, L = 2, 16, 16
NW = NC * NS
NPAIRS = F * (F - 1) // 2
GRPS = B // L
RPT = B // NS
BANDW = 1152
TOTALP = 26112

_ROW_START = [r * (F - 1) - r * (r - 1) // 2 for r in range(F - 1)]

_mesh = plsc.VectorSubcoreMesh(core_axis_name="c", subcore_axis_name="s")


@functools.partial(
    pl.kernel,
    out_type=(jax.ShapeDtypeStruct((NW, B), jnp.float32),
              jax.ShapeDtypeStruct((NC, F * B), jnp.int32)),
    mesh=_mesh,
    scratch_types=[
        pltpu.VMEM((D, BANDW), jnp.float32),
        pltpu.VMEM((D, BANDW), jnp.float32),
        pltpu.VMEM((D, BANDW), jnp.float32),
        pltpu.VMEM((D, BANDW), jnp.float32),
        pltpu.VMEM((B,), jnp.int32),
        pltpu.VMEM((B,), jnp.int32),
        pltpu.VMEM((B,), jnp.int32),
        pltpu.VMEM((B,), jnp.int32),
        pltpu.VMEM((B,), jnp.float32),
        pltpu.VMEM((ROWS,), jnp.float32),
        pltpu.VMEM((RPT * F,), jnp.int32),
        pltpu.VMEM((F * RPT,), jnp.int32),
        pltpu.SemaphoreType.DMA,
        pltpu.SemaphoreType.DMA,
    ],
    compiler_params=pltpu.CompilerParams(needs_layout_passes=False),
)
def _sc_ffm(ffm2, x_flat, lin_flat, out_ref, xt_hbm,
            band_a0, band_b0, band_a1, band_b1,
            x_a0, x_b0, x_a1, x_b1, partial, lin_band,
            x_rows, xt_chunk, sem0, sem1):
    cid = lax.axis_index("c")
    sid = lax.axis_index("s")
    wid = sid * NC + cid

    pltpu.sync_copy(x_flat.at[pl.ds(sid * (RPT * F), RPT * F)], x_rows)

    lanes = lax.iota(jnp.int32, L)

    @plsc.parallel_loop(0, F * (RPT // L), unroll=4)
    def t_body(q):
        f = q >> 4
        gg = q & 15
        idx = (gg * L + lanes) * F + f
        v = plsc.load_gather(x_rows, [idx])
        xt_chunk[pl.ds(f * RPT + gg * L, L)] = v

    for f in range(F):
        pltpu.sync_copy(xt_chunk.at[pl.ds(f * RPT, RPT)],
                        xt_hbm.at[cid, pl.ds(f * B + sid * RPT, RPT)])
    plsc.subcore_barrier()

    @plsc.parallel_loop(0, GRPS, unroll=4)
    def zero_body(g):
        partial[pl.ds(g * L, L)] = jnp.zeros((L,), jnp.float32)

    @pl.when(wid < F)
    def _():
        pltpu.sync_copy(lin_flat.at[pl.ds(wid * ROWS, ROWS)], lin_band)
        pltpu.sync_copy(xt_hbm.at[cid, pl.ds(wid * B, B)], x_a0)

        @plsc.parallel_loop(0, GRPS, unroll=4)
        def lin_body(g):
            base = g * L
            xv = x_a0[pl.ds(base, L)]
            lv = plsc.load_gather(lin_band, [xv])
            partial[pl.ds(base, L)] = partial[pl.ds(base, L)] + lv

    def _decode(k):
        p = wid + NW * k
        i_sc = jnp.int32(0)
        si_sc = jnp.int32(0)
        for r in range(1, F - 1):
            sr = _ROW_START[r]
            cond = p >= sr
            i_sc = jnp.where(cond, jnp.int32(r), i_sc)
            si_sc = jnp.where(cond, jnp.int32(sr), si_sc)
        j_sc = p - si_sc + i_sc + 1
        c0_a = pl.multiple_of(lax.div(i_sc * ROWS, 128) * 128, 128)
        c0_b = pl.multiple_of(lax.div(j_sc * ROWS, 128) * 128, 128)
        r_a = pl.multiple_of(j_sc * D, 8)
        r_b = pl.multiple_of(i_sc * D, 8)
        return i_sc, j_sc, c0_a, c0_b, r_a, r_b

    def _issue(k, ba, bb, xa, xb, sem):
        i_sc, j_sc, c0_a, c0_b, r_a, r_b = _decode(k)
        return (
            pltpu.async_copy(
                ffm2.at[pl.ds(r_a, D), pl.ds(c0_a, BANDW)], ba, sem),
            pltpu.async_copy(
                ffm2.at[pl.ds(r_b, D), pl.ds(c0_b, BANDW)], bb, sem),
            pltpu.async_copy(xt_hbm.at[cid, pl.ds(i_sc * B, B)], xa, sem),
            pltpu.async_copy(xt_hbm.at[cid, pl.ds(j_sc * B, B)], xb, sem),
        )

    def _compute(k, ba, bb, xa, xb, w):
        i_sc, j_sc, c0_a, c0_b, _, _ = _decode(k)
        sh_a = i_sc * ROWS - c0_a
        sh_b = j_sc * ROWS - c0_b

        @plsc.parallel_loop(0, GRPS, unroll=2)
        def grp_body(g):
            base = g * L
            ra = xa[pl.ds(base, L)] + sh_a
            rb = xb[pl.ds(base, L)] + sh_b
            accs = [jnp.zeros((L,), jnp.float32) for _ in range(4)]
            for d in range(D):
                dv = jnp.full((L,), d, jnp.int32)
                va = plsc.load_gather(ba, [dv, ra])
                vb = plsc.load_gather(bb, [dv, rb])
                accs[d % 4] = accs[d % 4] + va * vb
            acc = (accs[0] + accs[1]) + (accs[2] + accs[3])
            partial[pl.ds(base, L)] = partial[pl.ds(base, L)] + acc * w

    slots = [(band_a0, band_b0, x_a0, x_b0, sem0),
             (band_a1, band_b1, x_a1, x_b1, sem1)]
    n_k = jnp.where(wid < NPAIRS - 10 * NW, 11, 10)
    maxk = 11
    kk = [jnp.minimum(jnp.int32(k), n_k - 1) for k in range(maxk)]
    ws = [jnp.where(jnp.int32(k) < n_k, jnp.float32(1), jnp.float32(0))
          for k in range(maxk)]

    handles = {0: _issue(kk[0], *slots[0])}
    for k in range(maxk):
        if k + 1 < maxk:
            handles[k + 1] = _issue(kk[k + 1], *slots[(k + 1) % 2])
        for h in handles.pop(k):
            h.wait()
        _compute(kk[k], *slots[k % 2][:4], ws[k])

    pltpu.sync_copy(partial, out_ref.at[wid])


def _combine_body(p_ref, b_ref, o_ref):
    o_ref[...] = jnp.sum(p_ref[...], axis=0, keepdims=True) + b_ref[...]


_combine = pl.pallas_call(
    _combine_body,
    out_shape=jax.ShapeDtypeStruct((1, B), jnp.float32),
)


def kernel(x, linear_w, linear_b, ffm_w):
    x_flat = x.reshape(-1)
    lin_flat = linear_w.reshape(-1)
    ffm2 = jnp.pad(ffm_w.transpose(0, 2, 1).reshape(F * D, TOTAL),
                   ((0, 0), (0, TOTALP - TOTAL)))
    partials, _ = _sc_ffm(ffm2, x_flat, lin_flat)
    out = _combine(partials, linear_b.reshape(1, 1))
    return out.reshape(B, 1)
